# baseline (speedup 1.0000x reference)
import functools

import jax
import jax.numpy as jnp
from jax import lax
from jax.experimental import pallas as pl
from jax.experimental.pallas import tpu as pltpu
from jax.experimental.pallas import tpu_sc as plsc

_NC = 2
_NS = 16
_NW = _NC * _NS
_CH = 128
_KR = 2
_NB = 3
_LA = 1


@functools.cache
def _build(V, D, n_chunks):
    rows = _KR * _CH
    per_w = n_chunks * rows
    n_loop = ((n_chunks - _NB - _LA) // _NB) * _NB
    n_epi = n_chunks - _NB - n_loop
    mesh = plsc.VectorSubcoreMesh(core_axis_name="c", subcore_axis_name="s")

    @functools.partial(
        pl.kernel,
        out_type=jax.ShapeDtypeStruct((_NW * per_w, D), jnp.float32),
        mesh=mesh,
        scratch_types=[
            pltpu.VMEM((n_chunks * rows,), jnp.int32),
            [pltpu.VMEM((rows, D), jnp.float32) for _ in range(_NB)],
            [pltpu.SemaphoreType.DMA for _ in range(_NB)],
            [pltpu.SemaphoreType.DMA for _ in range(_NB)],
        ],
    )
    def k(idx_hbm, table_hbm, out_hbm, idx_v, bufs, gsems, wsems):
        wid = lax.axis_index("s") * _NC + lax.axis_index("c")
        base = wid * per_w

        def gather(c, b):
            pltpu.async_copy(
                table_hbm.at[idx_v.at[pl.ds(c * rows, rows)]], bufs[b],
                gsems[b])

        def wait_gather(b):
            pltpu.make_async_copy(
                table_hbm.at[pl.ds(0, rows)], bufs[b], gsems[b]).wait()

        def write(c, b):
            pltpu.async_copy(
                bufs[b], out_hbm.at[pl.ds(base + c * rows, rows)], wsems[b])

        def wait_write(b):
            pltpu.make_async_copy(
                bufs[b], out_hbm.at[pl.ds(base, rows)], wsems[b]).wait()

        def step(c, b, refill_c, need_wwait):
            wait_gather(b)
            write(c, b)
            if refill_c is not None:
                b2 = (b + _LA) % _NB
                if need_wwait:
                    wait_write(b2)
                gather(refill_c, b2)

        pltpu.sync_copy(idx_hbm.at[wid], idx_v)
        for c in range(_LA):
            gather(c, c % _NB)
        for c in range(_NB):
            step(c, c % _NB, c + _LA, c >= _NB - _LA)

        @pl.loop(_NB, _NB + n_loop, step=_NB)
        def _(i):
            for b in range(_NB):
                step(i + b, b, i + b + _LA, True)

        for e in range(n_epi):
            c = _NB + n_loop + e
            rc = c + _LA
            step(c, c % _NB, rc if rc < n_chunks else None, True)
        for c in range(n_chunks - _NB, n_chunks):
            wait_write(c % _NB)

    return k


def kernel(seqs, table):
    B, H = seqs.shape
    V, D = table.shape
    flat = seqs.reshape(-1).astype(jnp.int32)
    n = flat.shape[0]
    assert n % (_NW * _KR * _CH) == 0
    n_chunks = n // (_NW * _KR * _CH)
    assert n_chunks >= _NB + _LA
    idx = flat.reshape(_NW, n_chunks * _KR * _CH)
    out = _build(V, D, n_chunks)(idx, table)
    return out.reshape(B, H, D)
